# baseline (speedup 1.0000x reference)
import functools

import jax
import jax.numpy as jnp
from jax import lax
from jax.experimental import pallas as pl
from jax.experimental.pallas import tpu as pltpu
from jax.experimental.pallas import tpu_sc as plsc

N = 10000
E = 320000
G = 64
D_IN = 128
D_H = 256
D_OUT = 128

N_PAD = 10240
TRASH = N
R_SPMEM = N_PAD
K = 112
C_L1 = 3072
CPT_L1 = 96
CPT_L23 = 192
SGRP = 8
UNROLL = 24


@functools.lru_cache(maxsize=None)
def _sc_agg(cpt):
    assert cpt % UNROLL == 0
    nt = cpt // UNROLL
    mesh = plsc.VectorSubcoreMesh(core_axis_name="c", subcore_axis_name="s")

    @functools.partial(
        pl.kernel,
        out_type=jax.ShapeDtypeStruct((2 * N_PAD, 128), jnp.float32),
        mesh=mesh,
        scratch_types=[
            pltpu.VMEM((SGRP, K), jnp.int32),
            pltpu.VMEM((SGRP, K), jnp.int32),
            pltpu.VMEM((SGRP, K), jnp.int32),
            pltpu.VMEM((SGRP, K), jnp.int32),
            pltpu.VMEM((SGRP, K), jnp.int32),
            pltpu.VMEM((SGRP, K), jnp.int32),
            pltpu.VMEM((3, K, 128), jnp.float32),
            pltpu.VMEM_SHARED((R_SPMEM, 128), jnp.float32),
            [pltpu.SemaphoreType.DMA] * 3,
            [pltpu.SemaphoreType.DMA] * 3,
            [pltpu.SemaphoreType.DMA] * 3,
        ],
    )
    def kern(table_h, src_h, dst_h, out_h, sidx0, sidx1, sidx2, didx0, didx1,
             didx2, rows, agg, gsem, ssem, isem):
        c = lax.axis_index("c")
        s = lax.axis_index("s")
        wid = c * 16 + s
        cbase = wid * cpt
        sidx = [sidx0, sidx1, sidx2]
        didx = [didx0, didx1, didx2]

        def stage_start(grp_buf, chunk0):
            pltpu.async_copy(src_h.at[pl.ds(chunk0, SGRP)], sidx[grp_buf], isem[grp_buf])
            pltpu.async_copy(dst_h.at[pl.ds(chunk0, SGRP)], didx[grp_buf], isem[grp_buf])

        def stage_wait(grp_buf):
            for _ in range(2):
                pltpu.make_async_copy(
                    src_h.at[pl.ds(cbase, SGRP)], sidx[grp_buf], isem[grp_buf]
                ).wait()

        def gather_start(grp_buf, row, buf):
            pltpu.async_copy(table_h.at[sidx[grp_buf].at[row]], rows.at[buf], gsem[buf])

        def gather_wait(buf):
            pltpu.make_async_copy(
                table_h.at[sidx[0].at[0]], rows.at[buf], gsem[buf]
            ).wait()

        def scatter_start(grp_buf, row, buf):
            pltpu.async_copy(rows.at[buf], agg.at[didx[grp_buf].at[row]], ssem[buf], add=True)

        def scatter_wait(buf):
            pltpu.make_async_copy(
                rows.at[buf], agg.at[didx[0].at[0]], ssem[buf]
            ).wait()

        zero16 = jnp.zeros((16,), jnp.float32)

        def zbody(i, _):
            for q in range(8):
                rows[0, i, pl.ds(q * 16, 16)] = zero16
            return 0

        lax.fori_loop(0, K, zbody, 0)
        for j in range(5):
            pltpu.sync_copy(rows.at[0], agg.at[pl.ds(s * 640 + j * K, K)])
        pltpu.sync_copy(rows.at[0, pl.ds(0, 80)], agg.at[pl.ds(s * 640 + 560, 80)])

        plsc.subcore_barrier()

        stage_start(0, cbase)
        stage_start(1, cbase + SGRP)
        stage_wait(0)
        gather_start(0, 0, 0)
        gather_start(0, 1, 1)

        def iter_body(t, _):
            for l in range(UNROLL):
                bcur = l % 3
                bn = (l + 2) % 3
                if l == 0:
                    @pl.when(t > 0)
                    def _():
                        scatter_wait(bn)
                    stage_start(2, cbase + t * UNROLL + 2 * SGRP)
                else:
                    scatter_wait(bn)
                if l == 8:
                    @pl.when(t < nt - 1)
                    def _():
                        stage_start(0, cbase + (t + 1) * UNROLL)
                if l == 16:
                    @pl.when(t < nt - 1)
                    def _():
                        stage_start(1, cbase + (t + 1) * UNROLL + SGRP)
                if l == 6:
                    stage_wait(1)
                if l == 14:
                    stage_wait(2)
                if l == 22:
                    @pl.when(t < nt - 1)
                    def _():
                        stage_wait(0)
                nxt = l + 2
                if nxt < UNROLL:
                    gather_start(nxt // SGRP, nxt % SGRP, bn)
                else:
                    @pl.when(t < nt - 1)
                    def _():
                        gather_start(0, nxt - UNROLL, bn)
                gather_wait(bcur)
                scatter_start(l // SGRP, l % SGRP, bcur)
            return 0

        lax.fori_loop(0, nt, iter_body, 0)
        scatter_wait((cpt - 1) % 3)

        plsc.subcore_barrier()

        pltpu.sync_copy(
            agg.at[pl.ds(s * 640, 640)],
            out_h.at[pl.ds(c * N_PAD + s * 640, 640)],
        )

    return kern


_BLK = 1024
_NBLK = N_PAD // _BLK


def _mlp1_body(x_ref, p_ref, w_ref, b_ref, o_ref):
    u = x_ref[...] + p_ref[0] + p_ref[1]
    h = jnp.dot(u, w_ref[...], preferred_element_type=jnp.float32) + b_ref[...]
    h = jnp.maximum(h, 0.0)
    o_ref[0] = h[:, :128]
    o_ref[1] = h[:, 128:]


def _tc_layer1(xp, p, w, b):
    return pl.pallas_call(
        _mlp1_body,
        grid=(_NBLK,),
        in_specs=[
            pl.BlockSpec((_BLK, D_IN), lambda i: (i, 0)),
            pl.BlockSpec((2, _BLK, 128), lambda i: (0, i, 0)),
            pl.BlockSpec((D_IN, D_H), lambda i: (0, 0)),
            pl.BlockSpec((1, D_H), lambda i: (0, 0)),
        ],
        out_specs=pl.BlockSpec((2, _BLK, 128), lambda i: (0, i, 0)),
        out_shape=jax.ShapeDtypeStruct((2, N_PAD, 128), jnp.float32),
    )(xp, p, w, b)


def _mlp23_body(h_ref, a_ref, w_ref, b_ref, o_ref):
    ua = h_ref[0] + a_ref[0]
    ub = h_ref[1] + a_ref[1]
    acc = jnp.dot(ua, w_ref[:128, :], preferred_element_type=jnp.float32)
    acc += jnp.dot(ub, w_ref[128:, :], preferred_element_type=jnp.float32)
    h = jnp.maximum(acc + b_ref[...], 0.0)
    o_ref[0] = h[:, :128]
    o_ref[1] = h[:, 128:]


def _tc_layer23(hp, a, w, b):
    return pl.pallas_call(
        _mlp23_body,
        grid=(_NBLK,),
        in_specs=[
            pl.BlockSpec((2, _BLK, 128), lambda i: (0, i, 0)),
            pl.BlockSpec((2, _BLK, 128), lambda i: (0, i, 0)),
            pl.BlockSpec((D_H, D_H), lambda i: (0, 0)),
            pl.BlockSpec((1, D_H), lambda i: (0, 0)),
        ],
        out_specs=pl.BlockSpec((2, _BLK, 128), lambda i: (0, i, 0)),
        out_shape=jax.ShapeDtypeStruct((2, N_PAD, 128), jnp.float32),
    )(hp, a, w, b)


def _pool_body(h_ref, b_ref, wl_ref, bl_ref, o_ref, acc, cnt):
    i = pl.program_id(0)

    @pl.when(i == 0)
    def _():
        acc[...] = jnp.zeros_like(acc)
        cnt[...] = jnp.zeros_like(cnt)

    bvec = b_ref[0]
    gids = jax.lax.broadcasted_iota(jnp.int32, (G, _BLK), 0)
    onehot = (gids == jnp.broadcast_to(bvec, (G, _BLK))).astype(jnp.float32)
    hcat = jnp.concatenate([h_ref[0], h_ref[1]], axis=1)
    acc[...] += jnp.dot(onehot, hcat, preferred_element_type=jnp.float32)
    cnt[...] += jnp.sum(onehot, axis=1, keepdims=True)

    @pl.when(i == _NBLK - 1)
    def _():
        inv = 1.0 / jnp.maximum(cnt[...], 1.0)
        pooled = acc[...] * inv
        out = jnp.dot(pooled, wl_ref[...], preferred_element_type=jnp.float32)
        o_ref[...] = jnp.maximum(out + bl_ref[...], 0.0)


def _tc_pool(hp, batch3d, wl, bl):
    return pl.pallas_call(
        _pool_body,
        grid=(_NBLK,),
        in_specs=[
            pl.BlockSpec((2, _BLK, 128), lambda i: (0, i, 0)),
            pl.BlockSpec((1, 1, _BLK), lambda i: (i, 0, 0)),
            pl.BlockSpec((D_H, D_OUT), lambda i: (0, 0)),
            pl.BlockSpec((1, D_OUT), lambda i: (0, 0)),
        ],
        out_specs=pl.BlockSpec((G, D_OUT), lambda i: (0, 0)),
        out_shape=jax.ShapeDtypeStruct((G, D_OUT), jnp.float32),
        scratch_shapes=[
            pltpu.VMEM((G, D_H), jnp.float32),
            pltpu.VMEM((G, 1), jnp.float32),
        ],
    )(hp, batch3d, wl, bl)


def kernel(x, edge_index, batch, W1, b1, W2, b2, W3, b3, Wl, bl):
    src = edge_index[0].astype(jnp.int32)
    dst = edge_index[1].astype(jnp.int32)
    pad = C_L1 * K - E
    src_p = jnp.concatenate([src, jnp.zeros((pad,), jnp.int32)]).reshape(C_L1, K)
    dst_p = jnp.concatenate([dst, jnp.full((pad,), TRASH, jnp.int32)]).reshape(C_L1, K)
    src_stack = jnp.concatenate([src_p, src_p + N_PAD], axis=0)
    dst_stack = jnp.concatenate([dst_p, dst_p], axis=0)

    xp = jnp.zeros((N_PAD, D_IN), jnp.float32).at[:N].set(x)
    batch_p = jnp.concatenate(
        [batch.astype(jnp.int32), jnp.full((N_PAD - N,), G, jnp.int32)]
    ).reshape(_NBLK, 1, _BLK)
    b1r = b1.reshape(1, D_H)
    b2r = b2.reshape(1, D_H)
    b3r = b3.reshape(1, D_H)
    blr = bl.reshape(1, D_OUT)

    agg1 = _sc_agg(CPT_L1)(xp, src_p, dst_p)
    h1 = _tc_layer1(xp, agg1.reshape(2, N_PAD, 128), W1, b1r)

    agg2 = _sc_agg(CPT_L23)(h1.reshape(2 * N_PAD, 128), src_stack, dst_stack)
    h2 = _tc_layer23(h1, agg2.reshape(2, N_PAD, 128), W2, b2r)

    agg3 = _sc_agg(CPT_L23)(h2.reshape(2 * N_PAD, 128), src_stack, dst_stack)
    h3 = _tc_layer23(h2, agg3.reshape(2, N_PAD, 128), W3, b3r)

    return _tc_pool(h3, batch_p, Wl, blr)


# speedup vs baseline: 1.0047x; 1.0047x over previous
import functools

import jax
import jax.numpy as jnp
from jax import lax
from jax.experimental import pallas as pl
from jax.experimental.pallas import tpu as pltpu
from jax.experimental.pallas import tpu_sc as plsc

N = 10000
E = 320000
G = 64
D_IN = 128
D_H = 256
D_OUT = 128

N_PAD = 10240
TRASH = N
R_SPMEM = N_PAD
K = 112
C_L1 = 3072
CPT_L1 = 96
CPT_L23 = 192
SGRP = 8
UNROLL = 24


@functools.lru_cache(maxsize=None)
def _sc_agg(cpt):
    assert cpt % UNROLL == 0
    nt = cpt // UNROLL
    mesh = plsc.VectorSubcoreMesh(core_axis_name="c", subcore_axis_name="s")

    @functools.partial(
        pl.kernel,
        out_type=jax.ShapeDtypeStruct((2 * N_PAD, 128), jnp.float32),
        mesh=mesh,
        scratch_types=[
            pltpu.VMEM((SGRP, K), jnp.int32),
            pltpu.VMEM((SGRP, K), jnp.int32),
            pltpu.VMEM((SGRP, K), jnp.int32),
            pltpu.VMEM((SGRP, K), jnp.int32),
            pltpu.VMEM((SGRP, K), jnp.int32),
            pltpu.VMEM((SGRP, K), jnp.int32),
            pltpu.VMEM((3, K, 128), jnp.float32),
            pltpu.VMEM_SHARED((R_SPMEM, 128), jnp.float32),
            [pltpu.SemaphoreType.DMA] * 3,
            [pltpu.SemaphoreType.DMA] * 3,
            [pltpu.SemaphoreType.DMA] * 3,
        ],
    )
    def kern(table_h, src_h, dst_h, out_h, sidx0, sidx1, sidx2, didx0, didx1,
             didx2, rows, agg, gsem, ssem, isem):
        c = lax.axis_index("c")
        s = lax.axis_index("s")
        wid = c * 16 + s
        cbase = wid * cpt
        sidx = [sidx0, sidx1, sidx2]
        didx = [didx0, didx1, didx2]

        def stage_start(grp_buf, chunk0):
            pltpu.async_copy(src_h.at[pl.ds(chunk0, SGRP)], sidx[grp_buf], isem[grp_buf])
            pltpu.async_copy(dst_h.at[pl.ds(chunk0, SGRP)], didx[grp_buf], isem[grp_buf])

        def stage_wait(grp_buf):
            for _ in range(2):
                pltpu.make_async_copy(
                    src_h.at[pl.ds(cbase, SGRP)], sidx[grp_buf], isem[grp_buf]
                ).wait()

        def gather_start(grp_buf, row, buf):
            pltpu.async_copy(table_h.at[sidx[grp_buf].at[row]], rows.at[buf], gsem[buf])

        def gather_wait(buf):
            pltpu.make_async_copy(
                table_h.at[sidx[0].at[0]], rows.at[buf], gsem[buf]
            ).wait()

        def scatter_start(grp_buf, row, buf):
            pltpu.async_copy(rows.at[buf], agg.at[didx[grp_buf].at[row]], ssem[buf], add=True)

        def scatter_wait(buf):
            pltpu.make_async_copy(
                rows.at[buf], agg.at[didx[0].at[0]], ssem[buf]
            ).wait()

        zero16 = jnp.zeros((16,), jnp.float32)

        def zbody(i, _):
            for q in range(8):
                rows[0, i, pl.ds(q * 16, 16)] = zero16
            return 0

        lax.fori_loop(0, K, zbody, 0)
        for j in range(5):
            pltpu.sync_copy(rows.at[0], agg.at[pl.ds(s * 640 + j * K, K)])
        pltpu.sync_copy(rows.at[0, pl.ds(0, 80)], agg.at[pl.ds(s * 640 + 560, 80)])

        plsc.subcore_barrier()

        stage_start(0, cbase)
        stage_start(1, cbase + SGRP)
        stage_wait(0)
        gather_start(0, 0, 0)
        gather_start(0, 1, 1)

        def iter_body(t, _):
            for l in range(UNROLL):
                bcur = l % 3
                bn = (l + 2) % 3
                if l == 0:
                    @pl.when(t > 0)
                    def _():
                        scatter_wait(bn)
                    stage_start(2, cbase + t * UNROLL + 2 * SGRP)
                else:
                    scatter_wait(bn)
                if l == 8:
                    @pl.when(t < nt - 1)
                    def _():
                        stage_start(0, cbase + (t + 1) * UNROLL)
                if l == 16:
                    @pl.when(t < nt - 1)
                    def _():
                        stage_start(1, cbase + (t + 1) * UNROLL + SGRP)
                if l == 6:
                    stage_wait(1)
                if l == 14:
                    stage_wait(2)
                if l == 22:
                    @pl.when(t < nt - 1)
                    def _():
                        stage_wait(0)
                nxt = l + 2
                if nxt < UNROLL:
                    gather_start(nxt // SGRP, nxt % SGRP, bn)
                else:
                    @pl.when(t < nt - 1)
                    def _():
                        gather_start(0, nxt - UNROLL, bn)
                gather_wait(bcur)
                scatter_start(l // SGRP, l % SGRP, bcur)
            return 0

        lax.fori_loop(0, nt, iter_body, 0)
        scatter_wait((cpt - 1) % 3)

        plsc.subcore_barrier()

        pltpu.sync_copy(
            agg.at[pl.ds(s * 640, 640)],
            out_h.at[pl.ds(c * N_PAD + s * 640, 640)],
        )

    return kern


_BLK = 1024
_NBLK = N_PAD // _BLK


def _mlp1_body(x_ref, p_ref, w_ref, b_ref, o_ref):
    u = x_ref[...] + p_ref[0] + p_ref[1]
    h = jnp.dot(u, w_ref[...], preferred_element_type=jnp.float32) + b_ref[...]
    h = jnp.maximum(h, 0.0)
    o_ref[0] = h[:, :128]
    o_ref[1] = h[:, 128:]


def _tc_layer1(xp, p, w, b):
    return pl.pallas_call(
        _mlp1_body,
        grid=(_NBLK,),
        in_specs=[
            pl.BlockSpec((_BLK, D_IN), lambda i: (i, 0)),
            pl.BlockSpec((2, _BLK, 128), lambda i: (0, i, 0)),
            pl.BlockSpec((D_IN, D_H), lambda i: (0, 0)),
            pl.BlockSpec((1, D_H), lambda i: (0, 0)),
        ],
        out_specs=pl.BlockSpec((2, _BLK, 128), lambda i: (0, i, 0)),
        out_shape=jax.ShapeDtypeStruct((2, N_PAD, 128), jnp.float32),
    )(xp, p, w, b)


def _mlp23_body(h_ref, a_ref, w_ref, b_ref, o_ref):
    ua = h_ref[0] + a_ref[0]
    ub = h_ref[1] + a_ref[1]
    acc = jnp.dot(ua, w_ref[:128, :], preferred_element_type=jnp.float32)
    acc += jnp.dot(ub, w_ref[128:, :], preferred_element_type=jnp.float32)
    h = jnp.maximum(acc + b_ref[...], 0.0)
    o_ref[0] = h[:, :128]
    o_ref[1] = h[:, 128:]


def _tc_layer23(hp, a, w, b):
    return pl.pallas_call(
        _mlp23_body,
        grid=(_NBLK,),
        in_specs=[
            pl.BlockSpec((2, _BLK, 128), lambda i: (0, i, 0)),
            pl.BlockSpec((2, _BLK, 128), lambda i: (0, i, 0)),
            pl.BlockSpec((D_H, D_H), lambda i: (0, 0)),
            pl.BlockSpec((1, D_H), lambda i: (0, 0)),
        ],
        out_specs=pl.BlockSpec((2, _BLK, 128), lambda i: (0, i, 0)),
        out_shape=jax.ShapeDtypeStruct((2, N_PAD, 128), jnp.float32),
    )(hp, a, w, b)


def _pool_body(h_ref, b_ref, wl_ref, bl_ref, o_ref, acc, cnt):
    i = pl.program_id(0)

    @pl.when(i == 0)
    def _():
        acc[...] = jnp.zeros_like(acc)
        cnt[...] = jnp.zeros_like(cnt)

    bvec = b_ref[0]
    gids = jax.lax.broadcasted_iota(jnp.int32, (G, _BLK), 0)
    onehot = (gids == jnp.broadcast_to(bvec, (G, _BLK))).astype(jnp.float32)
    hcat = jnp.concatenate([h_ref[0], h_ref[1]], axis=1)
    acc[...] += jnp.dot(onehot, hcat, preferred_element_type=jnp.float32)
    cnt[...] += jnp.sum(onehot, axis=1, keepdims=True)

    @pl.when(i == _NBLK - 1)
    def _():
        inv = 1.0 / jnp.maximum(cnt[...], 1.0)
        pooled = acc[...] * inv
        out = jnp.dot(pooled, wl_ref[...], preferred_element_type=jnp.float32)
        o_ref[...] = jnp.maximum(out + bl_ref[...], 0.0)


def _tc_pool(hp, batch3d, wl, bl):
    return pl.pallas_call(
        _pool_body,
        grid=(_NBLK,),
        in_specs=[
            pl.BlockSpec((2, _BLK, 128), lambda i: (0, i, 0)),
            pl.BlockSpec((1, 1, _BLK), lambda i: (i, 0, 0)),
            pl.BlockSpec((D_H, D_OUT), lambda i: (0, 0)),
            pl.BlockSpec((1, D_OUT), lambda i: (0, 0)),
        ],
        out_specs=pl.BlockSpec((G, D_OUT), lambda i: (0, 0)),
        out_shape=jax.ShapeDtypeStruct((G, D_OUT), jnp.float32),
        scratch_shapes=[
            pltpu.VMEM((G, D_H), jnp.float32),
            pltpu.VMEM((G, 1), jnp.float32),
        ],
    )(hp, batch3d, wl, bl)


def kernel(x, edge_index, batch, W1, b1, W2, b2, W3, b3, Wl, bl):
    src = edge_index[0].astype(jnp.int32)
    dst = edge_index[1].astype(jnp.int32)
    pad = C_L1 * K - E
    pad_dst = TRASH + jax.lax.rem(jnp.arange(pad, dtype=jnp.int32), N_PAD - N)
    src_p = jnp.concatenate([src, jnp.zeros((pad,), jnp.int32)]).reshape(C_L1, K)
    dst_p = jnp.concatenate([dst, pad_dst]).reshape(C_L1, K)
    src_stack = jnp.concatenate([src_p, src_p + N_PAD], axis=0)
    dst_stack = jnp.concatenate([dst_p, dst_p], axis=0)

    xp = jnp.zeros((N_PAD, D_IN), jnp.float32).at[:N].set(x)
    batch_p = jnp.concatenate(
        [batch.astype(jnp.int32), jnp.full((N_PAD - N,), G, jnp.int32)]
    ).reshape(_NBLK, 1, _BLK)
    b1r = b1.reshape(1, D_H)
    b2r = b2.reshape(1, D_H)
    b3r = b3.reshape(1, D_H)
    blr = bl.reshape(1, D_OUT)

    agg1 = _sc_agg(CPT_L1)(xp, src_p, dst_p)
    h1 = _tc_layer1(xp, agg1.reshape(2, N_PAD, 128), W1, b1r)

    agg2 = _sc_agg(CPT_L23)(h1.reshape(2 * N_PAD, 128), src_stack, dst_stack)
    h2 = _tc_layer23(h1, agg2.reshape(2, N_PAD, 128), W2, b2r)

    agg3 = _sc_agg(CPT_L23)(h2.reshape(2 * N_PAD, 128), src_stack, dst_stack)
    h3 = _tc_layer23(h2, agg3.reshape(2, N_PAD, 128), W3, b3r)

    return _tc_pool(h3, batch_p, Wl, blr)


# speedup vs baseline: 8.2780x; 8.2391x over previous
import functools

import jax
import jax.numpy as jnp
from jax import lax
from jax.experimental import pallas as pl
from jax.experimental.pallas import tpu as pltpu
from jax.experimental.pallas import tpu_sc as plsc

N = 10000
E = 320000
G = 64
D_IN = 128
D_H = 256
D_OUT = 128

N_PAD = 10240
TRASH = N
R_SPMEM = N_PAD
K = 112
C_L1 = 3072
CPT_L1 = 96
CPT_L23 = 192
SGRP = 8
UNROLL = 24


@functools.lru_cache(maxsize=None)
def _sc_agg(cpt):
    assert cpt % UNROLL == 0
    nt = cpt // UNROLL
    mesh = plsc.VectorSubcoreMesh(core_axis_name="c", subcore_axis_name="s")

    @functools.partial(
        pl.kernel,
        out_type=jax.ShapeDtypeStruct((2 * N_PAD, 128), jnp.float32),
        mesh=mesh,
        scratch_types=[
            pltpu.VMEM((SGRP, K), jnp.int32),
            pltpu.VMEM((SGRP, K), jnp.int32),
            pltpu.VMEM((SGRP, K), jnp.int32),
            pltpu.VMEM((SGRP, K), jnp.int32),
            pltpu.VMEM((SGRP, K), jnp.int32),
            pltpu.VMEM((SGRP, K), jnp.int32),
            pltpu.VMEM((3, K, 128), jnp.float32),
            pltpu.VMEM_SHARED((R_SPMEM, 128), jnp.float32),
            [pltpu.SemaphoreType.DMA] * 3,
            [pltpu.SemaphoreType.DMA] * 3,
            [pltpu.SemaphoreType.DMA] * 3,
        ],
    )
    def kern(table_h, src_h, dst_h, out_h, sidx0, sidx1, sidx2, didx0, didx1,
             didx2, rows, agg, gsem, ssem, isem):
        c = lax.axis_index("c")
        s = lax.axis_index("s")
        wid = c * 16 + s
        cbase = wid * cpt
        sidx = [sidx0, sidx1, sidx2]
        didx = [didx0, didx1, didx2]

        def stage_start(grp_buf, chunk0):
            pltpu.async_copy(src_h.at[pl.ds(chunk0, SGRP)], sidx[grp_buf], isem[grp_buf])
            pltpu.async_copy(dst_h.at[pl.ds(chunk0, SGRP)], didx[grp_buf], isem[grp_buf])

        def stage_wait(grp_buf):
            for _ in range(2):
                pltpu.make_async_copy(
                    src_h.at[pl.ds(cbase, SGRP)], sidx[grp_buf], isem[grp_buf]
                ).wait()

        def gather_start(grp_buf, row, buf):
            pltpu.async_copy(table_h.at[sidx[grp_buf].at[row]], rows.at[buf], gsem[buf])

        def gather_wait(buf):
            pltpu.make_async_copy(
                table_h.at[sidx[0].at[0]], rows.at[buf], gsem[buf]
            ).wait()

        def scatter_start(grp_buf, row, buf):
            pltpu.async_copy(rows.at[buf], agg.at[didx[grp_buf].at[row]], ssem[buf], add=True)

        def scatter_wait(buf):
            pltpu.make_async_copy(
                rows.at[buf], agg.at[didx[0].at[0]], ssem[buf]
            ).wait()

        zero16 = jnp.zeros((16,), jnp.float32)

        def zbody(i, _):
            for q in range(8):
                rows[0, i, pl.ds(q * 16, 16)] = zero16
            return 0

        lax.fori_loop(0, K, zbody, 0)
        for j in range(5):
            pltpu.sync_copy(rows.at[0], agg.at[pl.ds(s * 640 + j * K, K)])
        pltpu.sync_copy(rows.at[0, pl.ds(0, 80)], agg.at[pl.ds(s * 640 + 560, 80)])

        plsc.subcore_barrier()

        stage_start(0, cbase)
        stage_start(1, cbase + SGRP)
        stage_wait(0)
        gather_start(0, 0, 0)
        gather_start(0, 1, 1)

        def iter_body(t, _):
            for l in range(UNROLL):
                bcur = l % 3
                bn = (l + 2) % 3
                if l == 0:
                    @pl.when(t > 0)
                    def _():
                        scatter_wait(bn)
                    stage_start(2, cbase + t * UNROLL + 2 * SGRP)
                else:
                    scatter_wait(bn)
                if l == 8:
                    @pl.when(t < nt - 1)
                    def _():
                        stage_start(0, cbase + (t + 1) * UNROLL)
                if l == 16:
                    @pl.when(t < nt - 1)
                    def _():
                        stage_start(1, cbase + (t + 1) * UNROLL + SGRP)
                if l == 6:
                    stage_wait(1)
                if l == 14:
                    stage_wait(2)
                if l == 22:
                    @pl.when(t < nt - 1)
                    def _():
                        stage_wait(0)
                nxt = l + 2
                if nxt < UNROLL:
                    gather_start(nxt // SGRP, nxt % SGRP, bn)
                else:
                    @pl.when(t < nt - 1)
                    def _():
                        gather_start(0, nxt - UNROLL, bn)
                gather_wait(bcur)
                scatter_start(l // SGRP, l % SGRP, bcur)
            return 0

        lax.fori_loop(0, nt, iter_body, 0)
        scatter_wait((cpt - 1) % 3)

        plsc.subcore_barrier()

        pltpu.sync_copy(
            agg.at[pl.ds(s * 640, 640)],
            out_h.at[pl.ds(c * N_PAD + s * 640, 640)],
        )

    return kern


_BLK = 1024
_NBLK = N_PAD // _BLK


def _mlp1_body(x_ref, p_ref, w_ref, b_ref, o_ref):
    u = x_ref[...] + p_ref[0] + p_ref[1]
    h = jnp.dot(u, w_ref[...], preferred_element_type=jnp.float32) + b_ref[...]
    h = jnp.maximum(h, 0.0)
    o_ref[0] = h[:, :128]
    o_ref[1] = h[:, 128:]


def _tc_layer1(xp, p, w, b):
    return pl.pallas_call(
        _mlp1_body,
        grid=(_NBLK,),
        in_specs=[
            pl.BlockSpec((_BLK, D_IN), lambda i: (i, 0)),
            pl.BlockSpec((2, _BLK, 128), lambda i: (0, i, 0)),
            pl.BlockSpec((D_IN, D_H), lambda i: (0, 0)),
            pl.BlockSpec((1, D_H), lambda i: (0, 0)),
        ],
        out_specs=pl.BlockSpec((2, _BLK, 128), lambda i: (0, i, 0)),
        out_shape=jax.ShapeDtypeStruct((2, N_PAD, 128), jnp.float32),
    )(xp, p, w, b)


def _mlp23_body(h_ref, a_ref, w_ref, b_ref, o_ref):
    ua = h_ref[0] + a_ref[0]
    ub = h_ref[1] + a_ref[1]
    acc = jnp.dot(ua, w_ref[:128, :], preferred_element_type=jnp.float32)
    acc += jnp.dot(ub, w_ref[128:, :], preferred_element_type=jnp.float32)
    h = jnp.maximum(acc + b_ref[...], 0.0)
    o_ref[0] = h[:, :128]
    o_ref[1] = h[:, 128:]


def _tc_layer23(hp, a, w, b):
    return pl.pallas_call(
        _mlp23_body,
        grid=(_NBLK,),
        in_specs=[
            pl.BlockSpec((2, _BLK, 128), lambda i: (0, i, 0)),
            pl.BlockSpec((2, _BLK, 128), lambda i: (0, i, 0)),
            pl.BlockSpec((D_H, D_H), lambda i: (0, 0)),
            pl.BlockSpec((1, D_H), lambda i: (0, 0)),
        ],
        out_specs=pl.BlockSpec((2, _BLK, 128), lambda i: (0, i, 0)),
        out_shape=jax.ShapeDtypeStruct((2, N_PAD, 128), jnp.float32),
    )(hp, a, w, b)


def _pool_body(h_ref, b_ref, wl_ref, bl_ref, o_ref, acc, cnt):
    i = pl.program_id(0)

    @pl.when(i == 0)
    def _():
        acc[...] = jnp.zeros_like(acc)
        cnt[...] = jnp.zeros_like(cnt)

    bvec = b_ref[0]
    gids = jax.lax.broadcasted_iota(jnp.int32, (G, _BLK), 0)
    onehot = (gids == jnp.broadcast_to(bvec, (G, _BLK))).astype(jnp.float32)
    hcat = jnp.concatenate([h_ref[0], h_ref[1]], axis=1)
    acc[...] += jnp.dot(onehot, hcat, preferred_element_type=jnp.float32)
    cnt[...] += jnp.sum(onehot, axis=1, keepdims=True)

    @pl.when(i == _NBLK - 1)
    def _():
        inv = 1.0 / jnp.maximum(cnt[...], 1.0)
        pooled = acc[...] * inv
        out = jnp.dot(pooled, wl_ref[...], preferred_element_type=jnp.float32)
        o_ref[...] = jnp.maximum(out + bl_ref[...], 0.0)


def _tc_pool(hp, batch3d, wl, bl):
    return pl.pallas_call(
        _pool_body,
        grid=(_NBLK,),
        in_specs=[
            pl.BlockSpec((2, _BLK, 128), lambda i: (0, i, 0)),
            pl.BlockSpec((1, 1, _BLK), lambda i: (i, 0, 0)),
            pl.BlockSpec((D_H, D_OUT), lambda i: (0, 0)),
            pl.BlockSpec((1, D_OUT), lambda i: (0, 0)),
        ],
        out_specs=pl.BlockSpec((G, D_OUT), lambda i: (0, 0)),
        out_shape=jax.ShapeDtypeStruct((G, D_OUT), jnp.float32),
        scratch_shapes=[
            pltpu.VMEM((G, D_H), jnp.float32),
            pltpu.VMEM((G, 1), jnp.float32),
        ],
    )(hp, batch3d, wl, bl)


def kernel(x, edge_index, batch, W1, b1, W2, b2, W3, b3, Wl, bl):
    src = edge_index[0].astype(jnp.int32)
    dst = edge_index[1].astype(jnp.int32)
    pad = C_L1 * K - E
    pad_iota = jnp.arange(pad, dtype=jnp.int32)
    pad_src = jax.lax.rem(pad_iota * 131, N)
    pad_dst = TRASH + jax.lax.rem(pad_iota, N_PAD - N)
    src_p = jnp.concatenate([src, pad_src]).reshape(C_L1, K)
    dst_p = jnp.concatenate([dst, pad_dst]).reshape(C_L1, K)
    src_stack = jnp.concatenate([src_p, src_p + N_PAD], axis=0)
    dst_stack = jnp.concatenate([dst_p, dst_p], axis=0)

    xp = jnp.zeros((N_PAD, D_IN), jnp.float32).at[:N].set(x)
    batch_p = jnp.concatenate(
        [batch.astype(jnp.int32), jnp.full((N_PAD - N,), G, jnp.int32)]
    ).reshape(_NBLK, 1, _BLK)
    b1r = b1.reshape(1, D_H)
    b2r = b2.reshape(1, D_H)
    b3r = b3.reshape(1, D_H)
    blr = bl.reshape(1, D_OUT)

    agg1 = _sc_agg(CPT_L1)(xp, src_p, dst_p)
    h1 = _tc_layer1(xp, agg1.reshape(2, N_PAD, 128), W1, b1r)

    agg2 = _sc_agg(CPT_L23)(h1.reshape(2 * N_PAD, 128), src_stack, dst_stack)
    h2 = _tc_layer23(h1, agg2.reshape(2, N_PAD, 128), W2, b2r)

    agg3 = _sc_agg(CPT_L23)(h2.reshape(2 * N_PAD, 128), src_stack, dst_stack)
    h3 = _tc_layer23(h2, agg3.reshape(2, N_PAD, 128), W3, b3r)

    return _tc_pool(h3, batch_p, Wl, blr)


# speedup vs baseline: 8.4660x; 1.0227x over previous
import functools

import jax
import jax.numpy as jnp
from jax import lax
from jax.experimental import pallas as pl
from jax.experimental.pallas import tpu as pltpu
from jax.experimental.pallas import tpu_sc as plsc

N = 10000
E = 320000
G = 64
D_IN = 128
D_H = 256
D_OUT = 128

N_PAD = 10240
TRASH = N
R_SPMEM = N_PAD
K = 112
C_L1 = 3072
CPT_L1 = 96
CPT_L23 = 192
SGRP = 8
UNROLL = 24


@functools.lru_cache(maxsize=None)
def _sc_agg(cpt):
    assert cpt % UNROLL == 0
    nt = cpt // UNROLL
    mesh = plsc.VectorSubcoreMesh(core_axis_name="c", subcore_axis_name="s")

    @functools.partial(
        pl.kernel,
        out_type=jax.ShapeDtypeStruct((2 * N_PAD, 128), jnp.float32),
        mesh=mesh,
        scratch_types=[
            pltpu.VMEM((SGRP, K), jnp.int32),
            pltpu.VMEM((SGRP, K), jnp.int32),
            pltpu.VMEM((SGRP, K), jnp.int32),
            pltpu.VMEM((SGRP, K), jnp.int32),
            pltpu.VMEM((SGRP, K), jnp.int32),
            pltpu.VMEM((SGRP, K), jnp.int32),
            pltpu.VMEM((3, K, 128), jnp.float32),
            pltpu.VMEM_SHARED((R_SPMEM, 128), jnp.float32),
            [pltpu.SemaphoreType.DMA] * 3,
            [pltpu.SemaphoreType.DMA] * 3,
            [pltpu.SemaphoreType.DMA] * 3,
        ],
    )
    def kern(table_h, src_h, dst_h, out_h, sidx0, sidx1, sidx2, didx0, didx1,
             didx2, rows, agg, gsem, ssem, isem):
        c = lax.axis_index("c")
        s = lax.axis_index("s")
        wid = c * 16 + s
        cbase = wid * cpt
        sidx = [sidx0, sidx1, sidx2]
        didx = [didx0, didx1, didx2]

        def stage_start(grp_buf, chunk0):
            pltpu.async_copy(src_h.at[pl.ds(chunk0, SGRP)], sidx[grp_buf], isem[grp_buf])
            pltpu.async_copy(dst_h.at[pl.ds(chunk0, SGRP)], didx[grp_buf], isem[grp_buf])

        def stage_wait(grp_buf):
            for _ in range(2):
                pltpu.make_async_copy(
                    src_h.at[pl.ds(cbase, SGRP)], sidx[grp_buf], isem[grp_buf]
                ).wait()

        def gather_start(grp_buf, row, buf):
            pltpu.async_copy(table_h.at[sidx[grp_buf].at[row]], rows.at[buf], gsem[buf])

        def gather_wait(buf):
            pltpu.make_async_copy(
                table_h.at[sidx[0].at[0]], rows.at[buf], gsem[buf]
            ).wait()

        def scatter_start(grp_buf, row, buf):
            pltpu.async_copy(rows.at[buf], agg.at[didx[grp_buf].at[row]], ssem[buf], add=True)

        def scatter_wait(buf):
            pltpu.make_async_copy(
                rows.at[buf], agg.at[didx[0].at[0]], ssem[buf]
            ).wait()

        zero16 = jnp.zeros((16,), jnp.float32)

        def zbody(i, _):
            for q in range(8):
                rows[0, i, pl.ds(q * 16, 16)] = zero16
            return 0

        lax.fori_loop(0, K, zbody, 0)
        for j in range(5):
            pltpu.sync_copy(rows.at[0], agg.at[pl.ds(s * 640 + j * K, K)])
        pltpu.sync_copy(rows.at[0, pl.ds(0, 80)], agg.at[pl.ds(s * 640 + 560, 80)])

        plsc.subcore_barrier()

        stage_start(0, cbase)
        stage_start(1, cbase + SGRP)
        stage_wait(0)
        gather_start(0, 0, 0)
        gather_start(0, 1, 1)

        def iter_body(t, _):
            for l in range(UNROLL):
                bcur = l % 3
                bn = (l + 2) % 3
                if l == 0:
                    @pl.when(t > 0)
                    def _():
                        scatter_wait(bn)
                    stage_start(2, cbase + t * UNROLL + 2 * SGRP)
                else:
                    scatter_wait(bn)
                if l == 8:
                    @pl.when(t < nt - 1)
                    def _():
                        stage_start(0, cbase + (t + 1) * UNROLL)
                if l == 16:
                    @pl.when(t < nt - 1)
                    def _():
                        stage_start(1, cbase + (t + 1) * UNROLL + SGRP)
                if l == 6:
                    stage_wait(1)
                if l == 14:
                    stage_wait(2)
                if l == 22:
                    @pl.when(t < nt - 1)
                    def _():
                        stage_wait(0)
                nxt = l + 2
                if nxt < UNROLL:
                    gather_start(nxt // SGRP, nxt % SGRP, bn)
                else:
                    @pl.when(t < nt - 1)
                    def _():
                        gather_start(0, nxt - UNROLL, bn)
                gather_wait(bcur)
                scatter_start(l // SGRP, l % SGRP, bcur)
            return 0

        lax.fori_loop(0, nt, iter_body, 0)
        scatter_wait((cpt - 1) % 3)

        plsc.subcore_barrier()

        pltpu.sync_copy(
            agg.at[pl.ds(s * 640, 640)],
            out_h.at[pl.ds(c * N_PAD + s * 640, 640)],
        )

    return kern


_BLK = 1024
_NBLK = N_PAD // _BLK


def _mlp1_body(x_ref, p_ref, w_ref, b_ref, o_ref):
    u = x_ref[...] + p_ref[0] + p_ref[1]
    h = jnp.dot(u, w_ref[...], preferred_element_type=jnp.float32) + b_ref[...]
    h = jnp.maximum(h, 0.0)
    o_ref[0] = h[:, :128]
    o_ref[1] = h[:, 128:]


def _tc_layer1(xp, p, w, b):
    return pl.pallas_call(
        _mlp1_body,
        grid=(_NBLK,),
        in_specs=[
            pl.BlockSpec((_BLK, D_IN), lambda i: (i, 0)),
            pl.BlockSpec((2, _BLK, 128), lambda i: (0, i, 0)),
            pl.BlockSpec((D_IN, D_H), lambda i: (0, 0)),
            pl.BlockSpec((1, D_H), lambda i: (0, 0)),
        ],
        out_specs=pl.BlockSpec((2, _BLK, 128), lambda i: (0, i, 0)),
        out_shape=jax.ShapeDtypeStruct((2, N_PAD, 128), jnp.float32),
    )(xp, p, w, b)


def _mlp2_body(h_ref, a_ref, w_ref, b_ref, o_ref):
    ua = h_ref[0] + a_ref[0]
    ub = h_ref[1] + a_ref[1]
    acc = jnp.dot(ua, w_ref[:128, :], preferred_element_type=jnp.float32)
    acc += jnp.dot(ub, w_ref[128:, :], preferred_element_type=jnp.float32)
    h = jnp.maximum(acc + b_ref[...], 0.0)
    o_ref[0] = h[:, :128]
    o_ref[1] = h[:, 128:]


def _tc_layer2(hp, a, w, b):
    return pl.pallas_call(
        _mlp2_body,
        grid=(_NBLK,),
        in_specs=[
            pl.BlockSpec((2, _BLK, 128), lambda i: (0, i, 0)),
            pl.BlockSpec((2, _BLK, 128), lambda i: (0, i, 0)),
            pl.BlockSpec((D_H, D_H), lambda i: (0, 0)),
            pl.BlockSpec((1, D_H), lambda i: (0, 0)),
        ],
        out_specs=pl.BlockSpec((2, _BLK, 128), lambda i: (0, i, 0)),
        out_shape=jax.ShapeDtypeStruct((2, N_PAD, 128), jnp.float32),
    )(hp, a, w, b)


def _mlp3_pool_body(h_ref, a_ref, w_ref, b_ref, batch_ref, wl_ref, bl_ref,
                    o_ref, acc, cnt):
    i = pl.program_id(0)

    @pl.when(i == 0)
    def _():
        acc[...] = jnp.zeros_like(acc)
        cnt[...] = jnp.zeros_like(cnt)

    ua = h_ref[0] + a_ref[0]
    ub = h_ref[1] + a_ref[1]
    h = jnp.dot(ua, w_ref[:128, :], preferred_element_type=jnp.float32)
    h += jnp.dot(ub, w_ref[128:, :], preferred_element_type=jnp.float32)
    h = jnp.maximum(h + b_ref[...], 0.0)

    bvec = batch_ref[0]
    gids = jax.lax.broadcasted_iota(jnp.int32, (G, _BLK), 0)
    onehot = (gids == jnp.broadcast_to(bvec, (G, _BLK))).astype(jnp.float32)
    acc[...] += jnp.dot(onehot, h, preferred_element_type=jnp.float32)
    cnt[...] += jnp.sum(onehot, axis=1, keepdims=True)

    @pl.when(i == _NBLK - 1)
    def _():
        inv = 1.0 / jnp.maximum(cnt[...], 1.0)
        pooled = acc[...] * inv
        out = jnp.dot(pooled, wl_ref[...], preferred_element_type=jnp.float32)
        o_ref[...] = jnp.maximum(out + bl_ref[...], 0.0)


def _tc_layer3_pool(hp, a, w, b, batch3d, wl, bl):
    return pl.pallas_call(
        _mlp3_pool_body,
        grid=(_NBLK,),
        in_specs=[
            pl.BlockSpec((2, _BLK, 128), lambda i: (0, i, 0)),
            pl.BlockSpec((2, _BLK, 128), lambda i: (0, i, 0)),
            pl.BlockSpec((D_H, D_H), lambda i: (0, 0)),
            pl.BlockSpec((1, D_H), lambda i: (0, 0)),
            pl.BlockSpec((1, 1, _BLK), lambda i: (i, 0, 0)),
            pl.BlockSpec((D_H, D_OUT), lambda i: (0, 0)),
            pl.BlockSpec((1, D_OUT), lambda i: (0, 0)),
        ],
        out_specs=pl.BlockSpec((G, D_OUT), lambda i: (0, 0)),
        out_shape=jax.ShapeDtypeStruct((G, D_OUT), jnp.float32),
        scratch_shapes=[
            pltpu.VMEM((G, D_H), jnp.float32),
            pltpu.VMEM((G, 1), jnp.float32),
        ],
    )(hp, a, w, b, batch3d, wl, bl)


def kernel(x, edge_index, batch, W1, b1, W2, b2, W3, b3, Wl, bl):
    src = edge_index[0].astype(jnp.int32)
    dst = edge_index[1].astype(jnp.int32)
    pad = C_L1 * K - E
    pad_iota = jnp.arange(pad, dtype=jnp.int32)
    pad_src = jax.lax.rem(pad_iota * 131, N)
    pad_dst = TRASH + jax.lax.rem(pad_iota, N_PAD - N)
    src_p = jnp.concatenate([src, pad_src]).reshape(C_L1, K)
    dst_p = jnp.concatenate([dst, pad_dst]).reshape(C_L1, K)
    src_stack = jnp.concatenate([src_p, src_p + N_PAD], axis=0)
    dst_stack = jnp.concatenate([dst_p, dst_p], axis=0)

    xp = jnp.zeros((N_PAD, D_IN), jnp.float32).at[:N].set(x)
    batch_p = jnp.concatenate(
        [batch.astype(jnp.int32), jnp.full((N_PAD - N,), G, jnp.int32)]
    ).reshape(_NBLK, 1, _BLK)
    b1r = b1.reshape(1, D_H)
    b2r = b2.reshape(1, D_H)
    b3r = b3.reshape(1, D_H)
    blr = bl.reshape(1, D_OUT)

    agg1 = _sc_agg(CPT_L1)(xp, src_p, dst_p)
    h1 = _tc_layer1(xp, agg1.reshape(2, N_PAD, 128), W1, b1r)

    agg2 = _sc_agg(CPT_L23)(h1.reshape(2 * N_PAD, 128), src_stack, dst_stack)
    h2 = _tc_layer2(h1, agg2.reshape(2, N_PAD, 128), W2, b2r)

    agg3 = _sc_agg(CPT_L23)(h2.reshape(2 * N_PAD, 128), src_stack, dst_stack)
    return _tc_layer3_pool(
        h2, agg3.reshape(2, N_PAD, 128), W3, b3r, batch_p, Wl, blr
    )
